# initial kernel scaffold (unmeasured)
import jax
import jax.numpy as jnp
from jax import lax
from jax.experimental import pallas as pl
from jax.experimental.pallas import tpu as pltpu

N_DEV = 4
M_PER = 1024
NB = 1024
R = 4


def kernel(x, w_mat):
    m, k_loc = x.shape
    _, n = w_mat.shape

    def body(x_ref, w_ref, out_ref,
             comm_f, comm_b, wf_ref, wb_ref,
             send_f, recv_f, send_b, recv_b, lsem):
        i = lax.axis_index("i")
        right = lax.rem(i + 1, N_DEV)
        left = lax.rem(i + N_DEV - 1, N_DEV)

        barrier = pltpu.get_barrier_semaphore()
        for nbr in (left, right):
            pl.semaphore_signal(
                barrier, inc=1,
                device_id=(nbr,), device_id_type=pl.DeviceIdType.MESH,
            )
        pl.semaphore_wait(barrier, 2)

        def xchunk(c):
            return x_ref[pl.ds(c * M_PER, M_PER), :]

        def partial(c, w_blk):
            return jnp.dot(xchunk(c), w_blk[:, :],
                           preferred_element_type=jnp.float32)

        for r in range(R):
            qf = r
            qb = R + r

            cp_wf = pltpu.make_async_copy(
                w_ref.at[:, pl.ds(qf * NB, NB)], wf_ref, lsem.at[0])
            cp_wb = pltpu.make_async_copy(
                w_ref.at[:, pl.ds(qb * NB, NB)], wb_ref, lsem.at[1])
            cp_wf.start()
            cp_wb.start()
            cp_wf.wait()
            cp_wb.wait()

            s0 = (3 * r) % 2
            comm_f[s0, :, :] = partial(left, wf_ref)
            comm_b[s0, :, :] = partial(right, wb_ref)

            for s in range(3):
                h = 3 * r + s
                ss = h % 2
                rs = (h + 1) % 2
                rdma_f = pltpu.make_async_remote_copy(
                    src_ref=comm_f.at[ss], dst_ref=comm_f.at[rs],
                    send_sem=send_f.at[ss], recv_sem=recv_f.at[rs],
                    device_id=(right,), device_id_type=pl.DeviceIdType.MESH,
                )
                rdma_b = pltpu.make_async_remote_copy(
                    src_ref=comm_b.at[ss], dst_ref=comm_b.at[rs],
                    send_sem=send_b.at[ss], recv_sem=recv_b.at[rs],
                    device_id=(left,), device_id_type=pl.DeviceIdType.MESH,
                )
                rdma_f.start()
                rdma_b.start()
                rdma_f.wait()
                rdma_b.wait()

                cf = lax.rem(i + 2 * N_DEV - s - 2, N_DEV)
                cb = lax.rem(i + s + 2, N_DEV)
                comm_f[rs, :, :] = comm_f[rs, :, :] + partial(cf, wf_ref)
                comm_b[rs, :, :] = comm_b[rs, :, :] + partial(cb, wb_ref)

            fin = (3 * r + 3) % 2
            cp_of = pltpu.make_async_copy(
                comm_f.at[fin], out_ref.at[:, pl.ds(qf * NB, NB)], lsem.at[2])
            cp_ob = pltpu.make_async_copy(
                comm_b.at[fin], out_ref.at[:, pl.ds(qb * NB, NB)], lsem.at[3])
            cp_of.start()
            cp_ob.start()
            cp_of.wait()
            cp_ob.wait()

    out_shape = jax.ShapeDtypeStruct((M_PER, n), jnp.float32)
    return pl.pallas_call(
        body,
        out_shape=out_shape,
        in_specs=[
            pl.BlockSpec(memory_space=pltpu.VMEM),
            pl.BlockSpec(memory_space=pltpu.ANY),
        ],
        out_specs=pl.BlockSpec(memory_space=pltpu.ANY),
        scratch_shapes=[
            pltpu.VMEM((2, M_PER, NB), jnp.float32),
            pltpu.VMEM((2, M_PER, NB), jnp.float32),
            pltpu.VMEM((k_loc, NB), jnp.float32),
            pltpu.VMEM((k_loc, NB), jnp.float32),
            pltpu.SemaphoreType.DMA((2,)),
            pltpu.SemaphoreType.DMA((2,)),
            pltpu.SemaphoreType.DMA((2,)),
            pltpu.SemaphoreType.DMA((2,)),
            pltpu.SemaphoreType.DMA((4,)),
        ],
        compiler_params=pltpu.CompilerParams(collective_id=0),
    )(x, w_mat)


# baseline (device time: 695831 ns/iter reference)
import jax
import jax.numpy as jnp
from jax import lax
from jax.experimental import pallas as pl
from jax.experimental.pallas import tpu as pltpu

N_DEV = 4
M_PER = 1024
NB = 1024
R = 4


def kernel(x, w_mat):
    m, k_loc = x.shape
    _, n = w_mat.shape

    def body(x_ref, w_ref, out_ref,
             comm_f, comm_b, wf_ref, wb_ref,
             send_f, recv_f, send_b, recv_b, lsem):
        i = lax.axis_index("i")
        right = lax.rem(i + 1, N_DEV)
        left = lax.rem(i + N_DEV - 1, N_DEV)

        barrier = pltpu.get_barrier_semaphore()
        for nbr in (left, right):
            pl.semaphore_signal(
                barrier, inc=1,
                device_id=(nbr,), device_id_type=pl.DeviceIdType.MESH,
            )
        pl.semaphore_wait(barrier, 2)

        def xchunk(c):
            return x_ref[pl.ds(c * M_PER, M_PER), :]

        def partial(c, w_blk):
            return jnp.dot(xchunk(c), w_blk[:, :],
                           preferred_element_type=jnp.float32)

        for r in range(R):
            qf = r
            qb = R + r

            cp_wf = pltpu.make_async_copy(
                w_ref.at[:, pl.ds(qf * NB, NB)], wf_ref, lsem.at[0])
            cp_wb = pltpu.make_async_copy(
                w_ref.at[:, pl.ds(qb * NB, NB)], wb_ref, lsem.at[1])
            cp_wf.start()
            cp_wb.start()
            cp_wf.wait()
            cp_wb.wait()

            s0 = (3 * r) % 2
            comm_f[s0, :, :] = partial(left, wf_ref)
            comm_b[s0, :, :] = partial(right, wb_ref)

            for s in range(3):
                h = 3 * r + s
                ss = h % 2
                rs = (h + 1) % 2
                rdma_f = pltpu.make_async_remote_copy(
                    src_ref=comm_f.at[ss], dst_ref=comm_f.at[rs],
                    send_sem=send_f.at[ss], recv_sem=recv_f.at[rs],
                    device_id=(right,), device_id_type=pl.DeviceIdType.MESH,
                )
                rdma_b = pltpu.make_async_remote_copy(
                    src_ref=comm_b.at[ss], dst_ref=comm_b.at[rs],
                    send_sem=send_b.at[ss], recv_sem=recv_b.at[rs],
                    device_id=(left,), device_id_type=pl.DeviceIdType.MESH,
                )
                rdma_f.start()
                rdma_b.start()
                rdma_f.wait()
                rdma_b.wait()

                cf = lax.rem(i + 2 * N_DEV - s - 2, N_DEV)
                cb = lax.rem(i + s + 2, N_DEV)
                comm_f[rs, :, :] = comm_f[rs, :, :] + partial(cf, wf_ref)
                comm_b[rs, :, :] = comm_b[rs, :, :] + partial(cb, wb_ref)

            fin = (3 * r + 3) % 2
            cp_of = pltpu.make_async_copy(
                comm_f.at[fin], out_ref.at[:, pl.ds(qf * NB, NB)], lsem.at[2])
            cp_ob = pltpu.make_async_copy(
                comm_b.at[fin], out_ref.at[:, pl.ds(qb * NB, NB)], lsem.at[3])
            cp_of.start()
            cp_ob.start()
            cp_of.wait()
            cp_ob.wait()

    out_shape = jax.ShapeDtypeStruct((M_PER, n), jnp.float32)
    return pl.pallas_call(
        body,
        out_shape=out_shape,
        in_specs=[
            pl.BlockSpec(memory_space=pltpu.VMEM),
            pl.BlockSpec(memory_space=pl.ANY),
        ],
        out_specs=pl.BlockSpec(memory_space=pl.ANY),
        scratch_shapes=[
            pltpu.VMEM((2, M_PER, NB), jnp.float32),
            pltpu.VMEM((2, M_PER, NB), jnp.float32),
            pltpu.VMEM((k_loc, NB), jnp.float32),
            pltpu.VMEM((k_loc, NB), jnp.float32),
            pltpu.SemaphoreType.DMA((2,)),
            pltpu.SemaphoreType.DMA((2,)),
            pltpu.SemaphoreType.DMA((2,)),
            pltpu.SemaphoreType.DMA((2,)),
            pltpu.SemaphoreType.DMA((4,)),
        ],
        compiler_params=pltpu.CompilerParams(collective_id=0),
    )(x, w_mat)


# device time: 636725 ns/iter; 1.0928x vs baseline; 1.0928x over previous
import jax
import jax.numpy as jnp
from jax import lax
from jax.experimental import pallas as pl
from jax.experimental.pallas import tpu as pltpu

N_DEV = 4
M_PER = 1024
NB = 1024
R = 4


def kernel(x, w_mat):
    m, k_loc = x.shape
    _, n = w_mat.shape

    def body(x_ref, w_ref, out_ref,
             comm_f, comm_b, wf_ref, wb_ref, pf_ref, pb_ref,
             send_f, recv_f, send_b, recv_b, lsem):
        i = lax.axis_index("i")
        right = lax.rem(i + 1, N_DEV)
        left = lax.rem(i + N_DEV - 1, N_DEV)

        barrier = pltpu.get_barrier_semaphore()
        for nbr in (left, right):
            pl.semaphore_signal(
                barrier, inc=1,
                device_id=(nbr,), device_id_type=pl.DeviceIdType.MESH,
            )
        pl.semaphore_wait(barrier, 2)

        def xchunk(c):
            return x_ref[pl.ds(c * M_PER, M_PER), :]

        def partial(c, w_blk):
            return jnp.dot(xchunk(c), w_blk[:, :],
                           preferred_element_type=jnp.float32)

        def start_w_loads(r):
            wb_idx = r % 2
            cp_wf = pltpu.make_async_copy(
                w_ref.at[:, pl.ds(r * NB, NB)], wf_ref.at[wb_idx], lsem.at[0])
            cp_wb = pltpu.make_async_copy(
                w_ref.at[:, pl.ds((R + r) * NB, NB)], wb_ref.at[wb_idx],
                lsem.at[1])
            cp_wf.start()
            cp_wb.start()
            return cp_wf, cp_wb

        w_loads = start_w_loads(0)
        out_cps = None

        for r in range(R):
            qf = r
            qb = R + r
            wi = r % 2
            wf = wf_ref.at[wi]
            wb = wb_ref.at[wi]

            w_loads[0].wait()
            w_loads[1].wait()
            if out_cps is not None:
                out_cps[0].wait()
                out_cps[1].wait()

            s0 = (3 * r) % 2
            comm_f[s0, :, :] = partial(left, wf)
            comm_b[s0, :, :] = partial(right, wb)

            for s in range(3):
                h = 3 * r + s
                ss = h % 2
                rs = (h + 1) % 2
                rdma_f = pltpu.make_async_remote_copy(
                    src_ref=comm_f.at[ss], dst_ref=comm_f.at[rs],
                    send_sem=send_f.at[ss], recv_sem=recv_f.at[rs],
                    device_id=(right,), device_id_type=pl.DeviceIdType.MESH,
                )
                rdma_b = pltpu.make_async_remote_copy(
                    src_ref=comm_b.at[ss], dst_ref=comm_b.at[rs],
                    send_sem=send_b.at[ss], recv_sem=recv_b.at[rs],
                    device_id=(left,), device_id_type=pl.DeviceIdType.MESH,
                )
                rdma_f.start()
                rdma_b.start()

                cf = lax.rem(i + 2 * N_DEV - s - 2, N_DEV)
                cb = lax.rem(i + s + 2, N_DEV)
                pf_ref[:, :] = partial(cf, wf)
                pb_ref[:, :] = partial(cb, wb)
                if s == 2 and r + 1 < R:
                    w_loads = start_w_loads(r + 1)

                rdma_f.wait()
                rdma_b.wait()
                comm_f[rs, :, :] = comm_f[rs, :, :] + pf_ref[:, :]
                comm_b[rs, :, :] = comm_b[rs, :, :] + pb_ref[:, :]

            fin = (3 * r + 3) % 2
            cp_of = pltpu.make_async_copy(
                comm_f.at[fin], out_ref.at[:, pl.ds(qf * NB, NB)], lsem.at[2])
            cp_ob = pltpu.make_async_copy(
                comm_b.at[fin], out_ref.at[:, pl.ds(qb * NB, NB)], lsem.at[3])
            cp_of.start()
            cp_ob.start()
            out_cps = (cp_of, cp_ob)

        out_cps[0].wait()
        out_cps[1].wait()

    out_shape = jax.ShapeDtypeStruct((M_PER, n), jnp.float32)
    return pl.pallas_call(
        body,
        out_shape=out_shape,
        in_specs=[
            pl.BlockSpec(memory_space=pltpu.VMEM),
            pl.BlockSpec(memory_space=pl.ANY),
        ],
        out_specs=pl.BlockSpec(memory_space=pl.ANY),
        scratch_shapes=[
            pltpu.VMEM((2, M_PER, NB), jnp.float32),
            pltpu.VMEM((2, M_PER, NB), jnp.float32),
            pltpu.VMEM((2, k_loc, NB), jnp.float32),
            pltpu.VMEM((2, k_loc, NB), jnp.float32),
            pltpu.VMEM((M_PER, NB), jnp.float32),
            pltpu.VMEM((M_PER, NB), jnp.float32),
            pltpu.SemaphoreType.DMA((2,)),
            pltpu.SemaphoreType.DMA((2,)),
            pltpu.SemaphoreType.DMA((2,)),
            pltpu.SemaphoreType.DMA((2,)),
            pltpu.SemaphoreType.DMA((4,)),
        ],
        compiler_params=pltpu.CompilerParams(
            collective_id=0,
            vmem_limit_bytes=60 * 1024 * 1024,
        ),
    )(x, w_mat)


# device time: 584902 ns/iter; 1.1897x vs baseline; 1.0886x over previous
import jax
import jax.numpy as jnp
from jax import lax
from jax.experimental import pallas as pl
from jax.experimental.pallas import tpu as pltpu

N_DEV = 4
M_PER = 1024
NB = 1024
H = 512
R = 4
HOPS = 3 * R

SEND = [0, 1, 2, 1, 2, 0, 2, 0, 1, 0, 1, 2]
RECV = [1, 2, 0, 2, 0, 1, 0, 1, 2, 1, 2, 0]


def kernel(x, w_mat):
    m, k_loc = x.shape
    _, n = w_mat.shape

    def body(x_ref, w_ref, out_ref,
             comm_f, comm_b, wf_ref, wb_ref, pf_ref, pb_ref,
             send_f, recv_f, send_b, recv_b, lsem):
        i = lax.axis_index("i")
        right = lax.rem(i + 1, N_DEV)
        left = lax.rem(i + N_DEV - 1, N_DEV)

        barrier = pltpu.get_barrier_semaphore()
        for nbr in (left, right):
            pl.semaphore_signal(
                barrier, inc=1,
                device_id=(nbr,), device_id_type=pl.DeviceIdType.MESH,
            )
        pl.semaphore_wait(barrier, 2)

        def xchunk(c):
            return x_ref[pl.ds(c * M_PER, M_PER), :]

        def hop_rdma(fwd, h, half, src_slot=None):
            comm = comm_f if fwd else comm_b
            ssem = send_f if fwd else send_b
            rsem = recv_f if fwd else recv_b
            dev = right if fwd else left
            ss = SEND[h] if src_slot is None else src_slot
            return pltpu.make_async_remote_copy(
                src_ref=comm.at[ss, :, pl.ds(half * H, H)],
                dst_ref=comm.at[RECV[h], :, pl.ds(half * H, H)],
                send_sem=ssem.at[ss, half],
                recv_sem=rsem.at[RECV[h], half],
                device_id=(dev,), device_id_type=pl.DeviceIdType.MESH,
            )

        def start_w_loads(r):
            wb_idx = r % 2
            cp_wf = pltpu.make_async_copy(
                w_ref.at[:, pl.ds(r * NB, NB)], wf_ref.at[wb_idx], lsem.at[0])
            cp_wb = pltpu.make_async_copy(
                w_ref.at[:, pl.ds((R + r) * NB, NB)], wb_ref.at[wb_idx],
                lsem.at[1])
            cp_wf.start()
            cp_wb.start()
            return cp_wf, cp_wb

        w_loads = start_w_loads(0)
        w_loads[0].wait()
        w_loads[1].wait()
        comm_f[SEND[0], :, :] = jnp.dot(
            xchunk(left), wf_ref[0], preferred_element_type=jnp.float32)
        comm_b[SEND[0], :, :] = jnp.dot(
            xchunk(right), wb_ref[0], preferred_element_type=jnp.float32)
        for half in (0, 1):
            hop_rdma(True, 0, half).start()
            hop_rdma(False, 0, half).start()

        out_cps = None
        for h in range(HOPS):
            r, s = divmod(h, 3)
            wi = r % 2

            if h > 0:
                for half in (0, 1):
                    hop_rdma(True, h - 1, half).wait_send()
                    hop_rdma(False, h - 1, half).wait_send()

            if s == 0 and r + 1 < R:
                w_loads = start_w_loads(r + 1)

            cf = lax.rem(i + 2 * N_DEV - s - 2, N_DEV)
            cb = lax.rem(i + s + 2, N_DEV)

            for half in (0, 1):
                hc = pl.ds(half * H, H)
                pf_ref[:, :] = jnp.dot(
                    xchunk(cf), wf_ref[wi, :, hc],
                    preferred_element_type=jnp.float32)
                pb_ref[:, :] = jnp.dot(
                    xchunk(cb), wb_ref[wi, :, hc],
                    preferred_element_type=jnp.float32)

                hop_rdma(True, h, half).wait_recv()
                comm_f[RECV[h], :, hc] = comm_f[RECV[h], :, hc] + pf_ref[:, :]
                hop_rdma(False, h, half).wait_recv()
                comm_b[RECV[h], :, hc] = comm_b[RECV[h], :, hc] + pb_ref[:, :]
                if h + 1 < HOPS and s != 2:
                    hop_rdma(True, h + 1, half).start()
                    hop_rdma(False, h + 1, half).start()

                if s == 2 and half == 0 and r + 1 < R:
                    w_loads[0].wait()
                    w_loads[1].wait()
                    nwi = (r + 1) % 2
                    ns = SEND[h + 1]
                    comm_f[ns, :, :] = jnp.dot(
                        xchunk(left), wf_ref[nwi],
                        preferred_element_type=jnp.float32)
                    comm_b[ns, :, :] = jnp.dot(
                        xchunk(right), wb_ref[nwi],
                        preferred_element_type=jnp.float32)
                    for ihalf in (0, 1):
                        hop_rdma(True, h + 1, ihalf, src_slot=ns).start()
                        hop_rdma(False, h + 1, ihalf, src_slot=ns).start()

            if s == 2:
                if out_cps is not None:
                    out_cps[0].wait()
                    out_cps[1].wait()
                cp_of = pltpu.make_async_copy(
                    comm_f.at[RECV[h]], out_ref.at[:, pl.ds(r * NB, NB)],
                    lsem.at[2])
                cp_ob = pltpu.make_async_copy(
                    comm_b.at[RECV[h]], out_ref.at[:, pl.ds((R + r) * NB, NB)],
                    lsem.at[3])
                cp_of.start()
                cp_ob.start()
                out_cps = (cp_of, cp_ob)

        for half in (0, 1):
            hop_rdma(True, HOPS - 1, half).wait_send()
            hop_rdma(False, HOPS - 1, half).wait_send()
        out_cps[0].wait()
        out_cps[1].wait()

    out_shape = jax.ShapeDtypeStruct((M_PER, n), jnp.float32)
    return pl.pallas_call(
        body,
        out_shape=out_shape,
        in_specs=[
            pl.BlockSpec(memory_space=pltpu.VMEM),
            pl.BlockSpec(memory_space=pl.ANY),
        ],
        out_specs=pl.BlockSpec(memory_space=pl.ANY),
        scratch_shapes=[
            pltpu.VMEM((3, M_PER, NB), jnp.float32),
            pltpu.VMEM((3, M_PER, NB), jnp.float32),
            pltpu.VMEM((2, k_loc, NB), jnp.float32),
            pltpu.VMEM((2, k_loc, NB), jnp.float32),
            pltpu.VMEM((M_PER, H), jnp.float32),
            pltpu.VMEM((M_PER, H), jnp.float32),
            pltpu.SemaphoreType.DMA((3, 2)),
            pltpu.SemaphoreType.DMA((3, 2)),
            pltpu.SemaphoreType.DMA((3, 2)),
            pltpu.SemaphoreType.DMA((3, 2)),
            pltpu.SemaphoreType.DMA((4,)),
        ],
        compiler_params=pltpu.CompilerParams(
            collective_id=0,
            vmem_limit_bytes=62 * 1024 * 1024,
        ),
    )(x, w_mat)


# device time: 581381 ns/iter; 1.1969x vs baseline; 1.0061x over previous
import jax
import jax.numpy as jnp
from jax import lax
from jax.experimental import pallas as pl
from jax.experimental.pallas import tpu as pltpu

N_DEV = 4
M_PER = 1024
NB = 1024
H = 512
R = 4
HOPS = 3 * R

SEND = [0, 1, 2, 1, 2, 0, 2, 0, 1, 0, 1, 2]
RECV = [1, 2, 0, 2, 0, 1, 0, 1, 2, 1, 2, 0]


def kernel(x, w_mat):
    m, k_loc = x.shape
    _, n = w_mat.shape

    def body(x_ref, w_ref, out_ref,
             comm_f, comm_b, wf_ref, wb_ref, pf_ref, pb_ref,
             send_f, recv_f, send_b, recv_b, lsem):
        i = lax.axis_index("i")
        right = lax.rem(i + 1, N_DEV)
        left = lax.rem(i + N_DEV - 1, N_DEV)

        def start_w_loads(r):
            wb_idx = r % 2
            cp_wf = pltpu.make_async_copy(
                w_ref.at[:, pl.ds(r * NB, NB)], wf_ref.at[wb_idx], lsem.at[0])
            cp_wb = pltpu.make_async_copy(
                w_ref.at[:, pl.ds((R + r) * NB, NB)], wb_ref.at[wb_idx],
                lsem.at[1])
            cp_wf.start()
            cp_wb.start()
            return cp_wf, cp_wb

        w_loads = start_w_loads(0)

        barrier = pltpu.get_barrier_semaphore()
        for nbr in (left, right):
            pl.semaphore_signal(
                barrier, inc=1,
                device_id=(nbr,), device_id_type=pl.DeviceIdType.MESH,
            )
        pl.semaphore_wait(barrier, 2)

        def xchunk(c):
            return x_ref[pl.ds(c * M_PER, M_PER), :]

        def hop_rdma(fwd, h, half, src_slot=None):
            comm = comm_f if fwd else comm_b
            ssem = send_f if fwd else send_b
            rsem = recv_f if fwd else recv_b
            dev = right if fwd else left
            ss = SEND[h] if src_slot is None else src_slot
            return pltpu.make_async_remote_copy(
                src_ref=comm.at[ss, :, pl.ds(half * H, H)],
                dst_ref=comm.at[RECV[h], :, pl.ds(half * H, H)],
                send_sem=ssem.at[ss, half],
                recv_sem=rsem.at[RECV[h], half],
                device_id=(dev,), device_id_type=pl.DeviceIdType.MESH,
            )

        w_loads[0].wait()
        w_loads[1].wait()
        for half in (0, 1):
            hc = pl.ds(half * H, H)
            comm_f[SEND[0], :, hc] = jnp.dot(
                xchunk(left), wf_ref[0, :, hc],
                preferred_element_type=jnp.float32)
            comm_b[SEND[0], :, hc] = jnp.dot(
                xchunk(right), wb_ref[0, :, hc],
                preferred_element_type=jnp.float32)
            hop_rdma(True, 0, half).start()
            hop_rdma(False, 0, half).start()

        out_cps = None
        for h in range(HOPS):
            r, s = divmod(h, 3)
            wi = r % 2

            if h > 0:
                for half in (0, 1):
                    hop_rdma(True, h - 1, half).wait_send()
                    hop_rdma(False, h - 1, half).wait_send()

            if s == 0 and r + 1 < R:
                w_loads = start_w_loads(r + 1)

            cf = lax.rem(i + 2 * N_DEV - s - 2, N_DEV)
            cb = lax.rem(i + s + 2, N_DEV)

            for half in (0, 1):
                hc = pl.ds(half * H, H)
                pf_ref[:, :] = jnp.dot(
                    xchunk(cf), wf_ref[wi, :, hc],
                    preferred_element_type=jnp.float32)
                pb_ref[:, :] = jnp.dot(
                    xchunk(cb), wb_ref[wi, :, hc],
                    preferred_element_type=jnp.float32)

                hop_rdma(True, h, half).wait_recv()
                comm_f[RECV[h], :, hc] = comm_f[RECV[h], :, hc] + pf_ref[:, :]
                hop_rdma(False, h, half).wait_recv()
                comm_b[RECV[h], :, hc] = comm_b[RECV[h], :, hc] + pb_ref[:, :]
                if h + 1 < HOPS and s != 2:
                    hop_rdma(True, h + 1, half).start()
                    hop_rdma(False, h + 1, half).start()

                if s == 2 and half == 0 and r + 1 < R:
                    w_loads[0].wait()
                    w_loads[1].wait()
                    nwi = (r + 1) % 2
                    ns = SEND[h + 1]
                    comm_f[ns, :, :] = jnp.dot(
                        xchunk(left), wf_ref[nwi],
                        preferred_element_type=jnp.float32)
                    comm_b[ns, :, :] = jnp.dot(
                        xchunk(right), wb_ref[nwi],
                        preferred_element_type=jnp.float32)
                    for ihalf in (0, 1):
                        hop_rdma(True, h + 1, ihalf, src_slot=ns).start()
                        hop_rdma(False, h + 1, ihalf, src_slot=ns).start()

            if s == 2:
                if out_cps is not None:
                    out_cps[0].wait()
                    out_cps[1].wait()
                cp_of = pltpu.make_async_copy(
                    comm_f.at[RECV[h]], out_ref.at[:, pl.ds(r * NB, NB)],
                    lsem.at[2])
                cp_ob = pltpu.make_async_copy(
                    comm_b.at[RECV[h]], out_ref.at[:, pl.ds((R + r) * NB, NB)],
                    lsem.at[3])
                cp_of.start()
                cp_ob.start()
                out_cps = (cp_of, cp_ob)

        for half in (0, 1):
            hop_rdma(True, HOPS - 1, half).wait_send()
            hop_rdma(False, HOPS - 1, half).wait_send()
        out_cps[0].wait()
        out_cps[1].wait()

    out_shape = jax.ShapeDtypeStruct((M_PER, n), jnp.float32)
    return pl.pallas_call(
        body,
        out_shape=out_shape,
        in_specs=[
            pl.BlockSpec(memory_space=pltpu.VMEM),
            pl.BlockSpec(memory_space=pl.ANY),
        ],
        out_specs=pl.BlockSpec(memory_space=pl.ANY),
        scratch_shapes=[
            pltpu.VMEM((3, M_PER, NB), jnp.float32),
            pltpu.VMEM((3, M_PER, NB), jnp.float32),
            pltpu.VMEM((2, k_loc, NB), jnp.float32),
            pltpu.VMEM((2, k_loc, NB), jnp.float32),
            pltpu.VMEM((M_PER, H), jnp.float32),
            pltpu.VMEM((M_PER, H), jnp.float32),
            pltpu.SemaphoreType.DMA((3, 2)),
            pltpu.SemaphoreType.DMA((3, 2)),
            pltpu.SemaphoreType.DMA((3, 2)),
            pltpu.SemaphoreType.DMA((3, 2)),
            pltpu.SemaphoreType.DMA((4,)),
        ],
        compiler_params=pltpu.CompilerParams(
            collective_id=0,
            vmem_limit_bytes=62 * 1024 * 1024,
        ),
    )(x, w_mat)
